# baseline (speedup 1.0000x reference)
import functools

import jax
import jax.numpy as jnp
from jax import lax
from jax.experimental import pallas as pl
from jax.experimental.pallas import tpu as pltpu
from jax.experimental.pallas import tpu_sc as plsc

_N = 5000
_NP = 5120
_NBLK = _NP // 16
_K = 100
_KMAX = 128
_OUTW = 5
_OUTN = 512
_SCORE_THRESH = 0.05
_NMS_THRESH = 0.5


def _sc_body(x1h, y1h, x2h, y2h, sh, ordh, outh,
             x1v, y1v, x2v, y2v, sv, ordv, keep_all,
             kx1, ky1, kx2, ky2, kar, outv, sem):
    cid = lax.axis_index("c")
    sid = lax.axis_index("s")

    @pl.when(jnp.logical_and(cid == 0, sid == 0))
    def _():
        iota16 = lax.iota(jnp.int32, 16)
        zero16 = jnp.zeros((16,), jnp.int32)
        zerof = jnp.zeros((16,), jnp.float32)

        cps = [pltpu.async_copy(x1h, x1v, sem),
               pltpu.async_copy(y1h, y1v, sem),
               pltpu.async_copy(x2h, x2v, sem),
               pltpu.async_copy(y2h, y2v, sem),
               pltpu.async_copy(sh, sv, sem),
               pltpu.async_copy(ordh, ordv, sem)]

        for v in range(_KMAX // 16):
            kx1[pl.ds(v * 16, 16)] = zerof
            ky1[pl.ds(v * 16, 16)] = zerof
            kx2[pl.ds(v * 16, 16)] = zerof
            ky2[pl.ds(v * 16, 16)] = zerof
            kar[pl.ds(v * 16, 16)] = zerof
        for v in range(_OUTN // 16):
            outv[pl.ds(v * 16, 16)] = zerof

        for c in cps:
            c.wait()

        def gather_box(perm):
            bx1 = plsc.load_gather(x1v, [perm])
            by1 = plsc.load_gather(y1v, [perm])
            bx2 = plsc.load_gather(x2v, [perm])
            by2 = plsc.load_gather(y2v, [perm])
            return bx1, by1, bx2, by2

        def cond(carry):
            b, cnt = carry
            return jnp.logical_and(b < _NBLK, cnt < _K)

        def body(carry):
            b, cnt = carry
            gb = b * 16
            perm = plsc.load_gather(ordv, [gb + iota16])
            bx1, by1, bx2, by2 = gather_box(perm)
            sb = plsc.load_gather(sv, [perm])
            barea = (jnp.maximum(bx2 - bx1, 0.0)
                     * jnp.maximum(by2 - by1, 0.0))
            kv0 = jnp.where((sb > _SCORE_THRESH) & (gb + iota16 < _N),
                            1.0, 0.0)

            bxmin = jnp.min(bx1)
            bymin = jnp.min(by1)
            bxmax = jnp.max(bx2)
            bymax = jnp.max(by2)

            def resolve_grp(g, kv):
                jj = g * 16 + iota16
                xj1 = plsc.load_gather(kx1, [jj])
                yj1 = plsc.load_gather(ky1, [jj])
                xj2 = plsc.load_gather(kx2, [jj])
                yj2 = plsc.load_gather(ky2, [jj])
                ov = ((xj1 < bxmax) & (xj2 > bxmin)
                      & (yj1 < bymax) & (yj2 > bymin))
                any_ov = jnp.max(jnp.where(ov, 1.0, 0.0))

                def full(kv):
                    aj = plsc.load_gather(kar, [jj])
                    for u in range(16):
                        uu = zero16 + u
                        x1u = xj1.at[uu].get(mode="promise_in_bounds")
                        y1u = yj1.at[uu].get(mode="promise_in_bounds")
                        x2u = xj2.at[uu].get(mode="promise_in_bounds")
                        y2u = yj2.at[uu].get(mode="promise_in_bounds")
                        au = aj.at[uu].get(mode="promise_in_bounds")
                        w = jnp.maximum(jnp.minimum(x2u, bx2)
                                        - jnp.maximum(x1u, bx1), 0.0)
                        h = jnp.maximum(jnp.minimum(y2u, by2)
                                        - jnp.maximum(y1u, by1), 0.0)
                        inter = w * h
                        union = au + barea - inter
                        iou = inter / jnp.maximum(union, 1e-9)
                        sup = jnp.where(iou > _NMS_THRESH, 1.0, 0.0)
                        kv = kv * (1.0 - sup)
                    return kv

                return lax.cond(any_ov > 0.0, full, lambda k: k, kv)

            kv = lax.fori_loop(0, (cnt + 15) // 16, resolve_grp, kv0)

            for i in range(16):
                ii = zero16 + i
                ki = kv.at[ii].get(mode="promise_in_bounds")
                xi1 = bx1.at[ii].get(mode="promise_in_bounds")
                yi1 = by1.at[ii].get(mode="promise_in_bounds")
                xi2 = bx2.at[ii].get(mode="promise_in_bounds")
                yi2 = by2.at[ii].get(mode="promise_in_bounds")
                ai = (jnp.maximum(xi2 - xi1, 0.0)
                      * jnp.maximum(yi2 - yi1, 0.0))
                w = jnp.maximum(jnp.minimum(xi2, bx2)
                                - jnp.maximum(xi1, bx1), 0.0)
                h = jnp.maximum(jnp.minimum(yi2, by2)
                                - jnp.maximum(yi1, by1), 0.0)
                inter = w * h
                union = ai + barea - inter
                iou = inter / jnp.maximum(union, 1e-9)
                sup = jnp.where((iou > _NMS_THRESH) & (iota16 > i),
                                ki, 0.0)
                kv = kv * (1.0 - sup)

            plsc.store_scatter(keep_all, [gb + iota16], kv)
            kmask = kv > 0.0
            cum = plsc.cumsum(kv)
            slots = jnp.where(
                kmask, cnt.astype(jnp.float32) + cum - 1.0, 0.0
            ).astype(jnp.int32)
            plsc.store_scatter(kx1, [slots], bx1, mask=kmask)
            plsc.store_scatter(ky1, [slots], by1, mask=kmask)
            plsc.store_scatter(kx2, [slots], bx2, mask=kmask)
            plsc.store_scatter(ky2, [slots], by2, mask=kmask)
            plsc.store_scatter(kar, [slots], barea, mask=kmask)
            nb = jnp.sum(kv).astype(jnp.int32)
            return (b + 1, cnt + nb)

        b_fin, cnt_fin = lax.while_loop(
            cond, body, (jnp.int32(0), jnp.int32(0)))

        pad_base = cnt_fin.astype(jnp.float32)

        def selblk(bb, carry):
            kc, nc = carry
            idx = bb * 16 + iota16
            m = plsc.load_gather(keep_all, [idx])
            perm = plsc.load_gather(ordv, [idx])
            x1b, y1b, x2b, y2b = gather_box(perm)
            sb = plsc.load_gather(sv, [perm])
            kmask = m > 0.0
            kf = jnp.where(kmask, 1.0, 0.0)
            cum = plsc.cumsum(kf)
            slots_f = kc + cum - 1.0
            wm = kmask & (slots_f < jnp.float32(_K))
            slots = jnp.where(wm, slots_f, 0.0).astype(jnp.int32) * _OUTW
            plsc.store_scatter(outv, [slots + 0], x1b, mask=wm)
            plsc.store_scatter(outv, [slots + 1], y1b, mask=wm)
            plsc.store_scatter(outv, [slots + 2], x2b, mask=wm)
            plsc.store_scatter(outv, [slots + 3], y2b, mask=wm)
            plsc.store_scatter(outv, [slots + 4], sb, mask=wm)
            nkf = jnp.where((~kmask) & (idx < _N), 1.0, 0.0)
            ncum = plsc.cumsum(nkf)
            nslots_f = pad_base + nc + ncum - 1.0
            nwm = (nkf > 0.0) & (nslots_f < jnp.float32(_K))
            nslots = (jnp.where(nwm, nslots_f, 0.0).astype(jnp.int32)
                      * _OUTW)
            plsc.store_scatter(outv, [nslots + 0], x1b, mask=nwm)
            plsc.store_scatter(outv, [nslots + 1], y1b, mask=nwm)
            plsc.store_scatter(outv, [nslots + 2], x2b, mask=nwm)
            plsc.store_scatter(outv, [nslots + 3], y2b, mask=nwm)
            return (kc + jnp.sum(kf), nc + jnp.sum(nkf))

        lax.fori_loop(0, b_fin, selblk, (jnp.float32(0.0),
                                         jnp.float32(0.0)))
        pltpu.sync_copy(outv, outh)


_mesh = plsc.VectorSubcoreMesh(core_axis_name="c", subcore_axis_name="s")

_sc_call = functools.partial(
    pl.kernel,
    mesh=_mesh,
    out_type=jax.ShapeDtypeStruct((_OUTN,), jnp.float32),
    compiler_params=pltpu.CompilerParams(needs_layout_passes=False),
    scratch_types=[
        pltpu.VMEM((_NP,), jnp.float32),
        pltpu.VMEM((_NP,), jnp.float32),
        pltpu.VMEM((_NP,), jnp.float32),
        pltpu.VMEM((_NP,), jnp.float32),
        pltpu.VMEM((_NP,), jnp.float32),
        pltpu.VMEM((_NP,), jnp.int32),
        pltpu.VMEM((_NP,), jnp.float32),
        pltpu.VMEM((_KMAX,), jnp.float32),
        pltpu.VMEM((_KMAX,), jnp.float32),
        pltpu.VMEM((_KMAX,), jnp.float32),
        pltpu.VMEM((_KMAX,), jnp.float32),
        pltpu.VMEM((_KMAX,), jnp.float32),
        pltpu.VMEM((_OUTN,), jnp.float32),
        pltpu.SemaphoreType.DMA,
    ],
)(_sc_body)


def kernel(boxes, scores):
    order = jnp.argsort(-scores)
    order_p = jnp.concatenate(
        [order, jnp.arange(_N, _NP, dtype=order.dtype)]).astype(jnp.int32)
    bp = jnp.pad(boxes, ((0, _NP - _N), (0, 0)))
    sp = jnp.pad(scores, (0, _NP - _N))
    out = _sc_call(bp[:, 0], bp[:, 1], bp[:, 2], bp[:, 3], sp, order_p)
    return out[:_K * _OUTW].reshape(_K, _OUTW)


# speedup vs baseline: 1.0293x; 1.0293x over previous
import functools

import jax
import jax.numpy as jnp
from jax import lax
from jax.experimental import pallas as pl
from jax.experimental.pallas import tpu as pltpu
from jax.experimental.pallas import tpu_sc as plsc

_N = 5000
_NP = 5120
_NBLK = _NP // 16
_K = 100
_KMAX = 128
_OUTW = 5
_OUTN = 512
_SCORE_THRESH = 0.05
_NMS_THRESH = 0.5


def _sc_body(x1h, y1h, x2h, y2h, sh, ordh, outh,
             x1v, y1v, x2v, y2v, sv, ordv, keep_all,
             kx1, ky1, kx2, ky2, kar, outv, sem):
    cid = lax.axis_index("c")
    sid = lax.axis_index("s")

    @pl.when(jnp.logical_and(cid == 0, sid == 0))
    def _():
        iota16 = lax.iota(jnp.int32, 16)
        zero16 = jnp.zeros((16,), jnp.int32)
        zerof = jnp.zeros((16,), jnp.float32)

        cps = [pltpu.async_copy(x1h, x1v, sem),
               pltpu.async_copy(y1h, y1v, sem),
               pltpu.async_copy(x2h, x2v, sem),
               pltpu.async_copy(y2h, y2v, sem),
               pltpu.async_copy(sh, sv, sem),
               pltpu.async_copy(ordh, ordv, sem)]

        for v in range(_KMAX // 16):
            kx1[pl.ds(v * 16, 16)] = zerof
            ky1[pl.ds(v * 16, 16)] = zerof
            kx2[pl.ds(v * 16, 16)] = zerof
            ky2[pl.ds(v * 16, 16)] = zerof
            kar[pl.ds(v * 16, 16)] = zerof
        for v in range(_OUTN // 16):
            outv[pl.ds(v * 16, 16)] = zerof

        for c in cps:
            c.wait()

        def gather_box(perm):
            bx1 = plsc.load_gather(x1v, [perm])
            by1 = plsc.load_gather(y1v, [perm])
            bx2 = plsc.load_gather(x2v, [perm])
            by2 = plsc.load_gather(y2v, [perm])
            return bx1, by1, bx2, by2

        def cond(carry):
            b, cnt = carry
            return jnp.logical_and(b < _NBLK, cnt < _K)

        def body(carry):
            b, cnt = carry
            gb = b * 16
            perm = plsc.load_gather(ordv, [gb + iota16])
            bx1, by1, bx2, by2 = gather_box(perm)
            sb = plsc.load_gather(sv, [perm])
            barea = (jnp.maximum(bx2 - bx1, 0.0)
                     * jnp.maximum(by2 - by1, 0.0))
            kv0 = jnp.where((sb > _SCORE_THRESH) & (gb + iota16 < _N),
                            1.0, 0.0)

            def resolve4(g, kv):
                for u in range(4):
                    jj = zero16 + (g * 4 + u)
                    xj1 = plsc.load_gather(kx1, [jj])
                    yj1 = plsc.load_gather(ky1, [jj])
                    xj2 = plsc.load_gather(kx2, [jj])
                    yj2 = plsc.load_gather(ky2, [jj])
                    aj = plsc.load_gather(kar, [jj])
                    w = jnp.maximum(jnp.minimum(xj2, bx2)
                                    - jnp.maximum(xj1, bx1), 0.0)
                    h = jnp.maximum(jnp.minimum(yj2, by2)
                                    - jnp.maximum(yj1, by1), 0.0)
                    inter = w * h
                    union = aj + barea - inter
                    iou = inter / jnp.maximum(union, 1e-9)
                    sup = jnp.where(iou > _NMS_THRESH, 1.0, 0.0)
                    kv = kv * (1.0 - sup)
                return kv

            kv = lax.fori_loop(0, (cnt + 3) // 4, resolve4, kv0)

            for i in range(16):
                ii = zero16 + i
                ki = kv.at[ii].get(mode="promise_in_bounds")
                xi1 = bx1.at[ii].get(mode="promise_in_bounds")
                yi1 = by1.at[ii].get(mode="promise_in_bounds")
                xi2 = bx2.at[ii].get(mode="promise_in_bounds")
                yi2 = by2.at[ii].get(mode="promise_in_bounds")
                ai = (jnp.maximum(xi2 - xi1, 0.0)
                      * jnp.maximum(yi2 - yi1, 0.0))
                w = jnp.maximum(jnp.minimum(xi2, bx2)
                                - jnp.maximum(xi1, bx1), 0.0)
                h = jnp.maximum(jnp.minimum(yi2, by2)
                                - jnp.maximum(yi1, by1), 0.0)
                inter = w * h
                union = ai + barea - inter
                iou = inter / jnp.maximum(union, 1e-9)
                sup = jnp.where((iou > _NMS_THRESH) & (iota16 > i),
                                ki, 0.0)
                kv = kv * (1.0 - sup)

            plsc.store_scatter(keep_all, [gb + iota16], kv)
            kmask = kv > 0.0
            cum = plsc.cumsum(kv)
            slots = jnp.where(
                kmask, cnt.astype(jnp.float32) + cum - 1.0, 0.0
            ).astype(jnp.int32)
            plsc.store_scatter(kx1, [slots], bx1, mask=kmask)
            plsc.store_scatter(ky1, [slots], by1, mask=kmask)
            plsc.store_scatter(kx2, [slots], bx2, mask=kmask)
            plsc.store_scatter(ky2, [slots], by2, mask=kmask)
            plsc.store_scatter(kar, [slots], barea, mask=kmask)
            nb = jnp.sum(kv).astype(jnp.int32)
            return (b + 1, cnt + nb)

        b_fin, cnt_fin = lax.while_loop(
            cond, body, (jnp.int32(0), jnp.int32(0)))

        pad_base = cnt_fin.astype(jnp.float32)

        def selblk(bb, carry):
            kc, nc = carry
            idx = bb * 16 + iota16
            m = plsc.load_gather(keep_all, [idx])
            perm = plsc.load_gather(ordv, [idx])
            x1b, y1b, x2b, y2b = gather_box(perm)
            sb = plsc.load_gather(sv, [perm])
            kmask = m > 0.0
            kf = jnp.where(kmask, 1.0, 0.0)
            cum = plsc.cumsum(kf)
            slots_f = kc + cum - 1.0
            wm = kmask & (slots_f < jnp.float32(_K))
            slots = jnp.where(wm, slots_f, 0.0).astype(jnp.int32) * _OUTW
            plsc.store_scatter(outv, [slots + 0], x1b, mask=wm)
            plsc.store_scatter(outv, [slots + 1], y1b, mask=wm)
            plsc.store_scatter(outv, [slots + 2], x2b, mask=wm)
            plsc.store_scatter(outv, [slots + 3], y2b, mask=wm)
            plsc.store_scatter(outv, [slots + 4], sb, mask=wm)
            nkf = jnp.where((~kmask) & (idx < _N), 1.0, 0.0)
            ncum = plsc.cumsum(nkf)
            nslots_f = pad_base + nc + ncum - 1.0
            nwm = (nkf > 0.0) & (nslots_f < jnp.float32(_K))
            nslots = (jnp.where(nwm, nslots_f, 0.0).astype(jnp.int32)
                      * _OUTW)
            plsc.store_scatter(outv, [nslots + 0], x1b, mask=nwm)
            plsc.store_scatter(outv, [nslots + 1], y1b, mask=nwm)
            plsc.store_scatter(outv, [nslots + 2], x2b, mask=nwm)
            plsc.store_scatter(outv, [nslots + 3], y2b, mask=nwm)
            return (kc + jnp.sum(kf), nc + jnp.sum(nkf))

        lax.fori_loop(0, b_fin, selblk, (jnp.float32(0.0),
                                         jnp.float32(0.0)))
        pltpu.sync_copy(outv, outh)


_mesh = plsc.VectorSubcoreMesh(core_axis_name="c", subcore_axis_name="s")

_sc_call = functools.partial(
    pl.kernel,
    mesh=_mesh,
    out_type=jax.ShapeDtypeStruct((_OUTN,), jnp.float32),
    compiler_params=pltpu.CompilerParams(needs_layout_passes=False),
    scratch_types=[
        pltpu.VMEM((_NP,), jnp.float32),
        pltpu.VMEM((_NP,), jnp.float32),
        pltpu.VMEM((_NP,), jnp.float32),
        pltpu.VMEM((_NP,), jnp.float32),
        pltpu.VMEM((_NP,), jnp.float32),
        pltpu.VMEM((_NP,), jnp.int32),
        pltpu.VMEM((_NP,), jnp.float32),
        pltpu.VMEM((_KMAX,), jnp.float32),
        pltpu.VMEM((_KMAX,), jnp.float32),
        pltpu.VMEM((_KMAX,), jnp.float32),
        pltpu.VMEM((_KMAX,), jnp.float32),
        pltpu.VMEM((_KMAX,), jnp.float32),
        pltpu.VMEM((_OUTN,), jnp.float32),
        pltpu.SemaphoreType.DMA,
    ],
)(_sc_body)


def kernel(boxes, scores):
    order = jnp.argsort(-scores)
    order_p = jnp.concatenate(
        [order, jnp.arange(_N, _NP, dtype=order.dtype)]).astype(jnp.int32)
    bp = jnp.pad(boxes, ((0, _NP - _N), (0, 0)))
    sp = jnp.pad(scores, (0, _NP - _N))
    out = _sc_call(bp[:, 0], bp[:, 1], bp[:, 2], bp[:, 3], sp, order_p)
    return out[:_K * _OUTW].reshape(_K, _OUTW)
